# baseline (speedup 1.0000x reference)
import functools

import jax
import jax.numpy as jnp
from jax import lax
from jax.experimental import pallas as pl
from jax.experimental.pallas import tpu as pltpu
from jax.experimental.pallas import tpu_sc as plsc

_CHUNKS = 12


def _sc_block(x, codes_p, table, NC, NS):
    B, C, S = x.shape
    NP = codes_p.shape[-1]
    NW = NC * NS
    PB = B // NW
    mesh = plsc.VectorSubcoreMesh(core_axis_name="c", subcore_axis_name="s")

    chunk_plan = [(16 * k, 4 * k) for k in range(_CHUNKS)] + [(S - 16, (S - 16) // 4)]

    @functools.partial(
        pl.kernel,
        out_type=jax.ShapeDtypeStruct((B, 2 * C, S), jnp.float32),
        mesh=mesh,
        compiler_params=pltpu.CompilerParams(needs_layout_passes=False),
        scratch_types=[
            pltpu.VMEM((PB, NP), jnp.int32),
            pltpu.VMEM((2, NP, C), jnp.float32),
            pltpu.VMEM((2, C, S), jnp.float32),
            pltpu.SemaphoreType.DMA,
            pltpu.SemaphoreType.DMA,
            pltpu.SemaphoreType.DMA,
            pltpu.SemaphoreType.DMA,
            pltpu.SemaphoreType.DMA,
        ],
    )
    def body(x_hbm, idx_hbm, table_hbm, out_hbm, idx_v, emb_v, u_v,
             gsem0, gsem1, usem0, usem1, xsem):
        wid = lax.axis_index("s") * NC + lax.axis_index("c")
        b0 = wid * PB
        pltpu.sync_copy(idx_hbm.at[wid], idx_v)

        def fire_x(i, carry):
            pltpu.async_copy(x_hbm.at[b0 + i], out_hbm.at[b0 + i, pl.ds(0, C)],
                             xsem)
            return carry

        lax.fori_loop(0, PB, fire_x, 0)

        lanes = lax.iota(jnp.int32, 16)
        j4 = lax.shift_right_logical(lanes, 2)
        bases = [j4 + n0 for (_, n0) in chunk_plan]

        def issue_gather(bi, slot, sem):
            pltpu.async_copy(table_hbm.at[idx_v.at[bi]], emb_v.at[slot], sem)

        def wait_gather(bi, slot, sem):
            pltpu.make_async_copy(table_hbm.at[idx_v.at[bi]], emb_v.at[slot],
                                  sem).wait()

        def issue_u(bi, slot, sem):
            pltpu.async_copy(u_v.at[slot], out_hbm.at[b0 + bi, pl.ds(C, C)],
                             sem)

        def wait_u(bi, slot, sem):
            pltpu.make_async_copy(u_v.at[slot], out_hbm.at[b0 + bi, pl.ds(C, C)],
                                  sem).wait()

        def upsample(slot):
            ev = emb_v.at[slot]
            uv = u_v.at[slot]

            def cloop(c, carry):
                cvec = jnp.full((16,), c, jnp.int32)
                for k, (s0, _) in enumerate(chunk_plan):
                    vals = plsc.load_gather(ev, [bases[k], cvec])
                    uv[c, pl.ds(s0, 16)] = vals
                return carry

            lax.fori_loop(0, C, cloop, 0)

        issue_gather(0, 0, gsem0)

        def pair(bp, carry):
            bi0 = 2 * bp
            bi1 = bi0 + 1
            issue_gather(bi1, 1, gsem1)
            wait_gather(bi0, 0, gsem0)

            @pl.when(bp >= 1)
            def _():
                wait_u(bi0, 0, usem0)

            upsample(0)
            issue_u(bi0, 0, usem0)

            @pl.when(bp < PB // 2 - 1)
            def _():
                issue_gather(bi0 + 2, 0, gsem0)

            wait_gather(bi1, 1, gsem1)

            @pl.when(bp >= 1)
            def _():
                wait_u(bi1, 1, usem1)

            upsample(1)
            issue_u(bi1, 1, usem1)
            return carry

        lax.fori_loop(0, PB // 2, pair, 0)

        wait_u(PB - 2, 0, usem0)
        wait_u(PB - 1, 1, usem1)

        def drain_x(i, carry):
            pltpu.make_async_copy(x_hbm.at[b0 + i],
                                  out_hbm.at[b0 + i, pl.ds(0, C)], xsem).wait()
            return carry

        lax.fori_loop(0, PB, drain_x, 0)

    return body(x, codes_p, table)


def kernel(x, codes, W_emb):
    b, c, S = x.shape
    _, N = codes.shape

    info = plsc.get_sparse_core_info()
    NC, NS = info.num_cores, info.num_subcores
    NW = NC * NS

    NP = 56
    codes_p = jnp.pad(codes.astype(jnp.int32), ((0, 0), (0, NP - N)))
    codes_p = codes_p.reshape(NW, b // NW, NP)

    return _sc_block(x, codes_p, W_emb, NC, NS)


# speedup vs baseline: 2.8968x; 2.8968x over previous
import functools

import jax
import jax.numpy as jnp
from jax import lax
from jax.experimental import pallas as pl
from jax.experimental.pallas import tpu as pltpu
from jax.experimental.pallas import tpu_sc as plsc

_CHUNKS = 12


def _sc_block(x_flat, codes_p, table, B, C, S, NC, NS):
    NP = 56
    NW = NC * NS
    PB = B // NW
    XW = C * S
    OW = 2 * XW
    mesh = plsc.VectorSubcoreMesh(core_axis_name="c", subcore_axis_name="s")

    chunk_plan = [(16 * k, 4 * k) for k in range(_CHUNKS)] + [(S - 16, (S - 16) // 4)]

    @functools.partial(
        pl.kernel,
        out_type=jax.ShapeDtypeStruct((B * OW,), jnp.float32),
        mesh=mesh,
        compiler_params=pltpu.CompilerParams(needs_layout_passes=False),
        scratch_types=[
            pltpu.VMEM((PB * NP,), jnp.int32),
            pltpu.VMEM((NP, C), jnp.float32),
            pltpu.VMEM((NP, C), jnp.float32),
            pltpu.VMEM((XW,), jnp.float32),
            pltpu.VMEM((XW,), jnp.float32),
            pltpu.VMEM((XW,), jnp.float32),
            pltpu.VMEM((XW,), jnp.float32),
            pltpu.SemaphoreType.DMA,
            pltpu.SemaphoreType.DMA,
            pltpu.SemaphoreType.DMA,
            pltpu.SemaphoreType.DMA,
            pltpu.SemaphoreType.DMA,
            pltpu.SemaphoreType.DMA,
            pltpu.SemaphoreType.DMA,
            pltpu.SemaphoreType.DMA,
        ],
    )
    def body(x_hbm, idx_hbm, table_hbm, out_hbm, idx_v, emb_a, emb_b,
             u_a, u_b, x_a, x_b,
             gsem0, gsem1, usem0, usem1, xisem0, xisem1, xosem0, xosem1):
        wid = lax.axis_index("s") * NC + lax.axis_index("c")
        b0 = wid * PB
        embs = (emb_a, emb_b)
        us = (u_a, u_b)
        xs = (x_a, x_b)
        gsems = (gsem0, gsem1)
        usems = (usem0, usem1)
        xisems = (xisem0, xisem1)
        xosems = (xosem0, xosem1)
        pltpu.sync_copy(idx_hbm.at[pl.ds(wid * PB * NP, PB * NP)], idx_v)

        lanes = lax.iota(jnp.int32, 16)
        j4 = lax.shift_right_logical(lanes, 2)
        bases = [j4 + n0 for (_, n0) in chunk_plan]

        def issue_gather(bi, slot):
            pltpu.async_copy(table_hbm.at[idx_v.at[pl.ds(bi * NP, NP)]],
                             embs[slot], gsems[slot])

        def wait_gather(bi, slot):
            pltpu.make_async_copy(table_hbm.at[idx_v.at[pl.ds(bi * NP, NP)]],
                                  embs[slot], gsems[slot]).wait()

        def issue_xin(bi, slot):
            pltpu.async_copy(x_hbm.at[pl.ds((b0 + bi) * XW, XW)], xs[slot],
                             xisems[slot])

        def wait_xin(bi, slot):
            pltpu.make_async_copy(x_hbm.at[pl.ds((b0 + bi) * XW, XW)],
                                  xs[slot], xisems[slot]).wait()

        def issue_xout(bi, slot):
            pltpu.async_copy(xs[slot],
                             out_hbm.at[pl.ds((b0 + bi) * OW, XW)],
                             xosems[slot])

        def wait_xout(bi, slot):
            pltpu.make_async_copy(xs[slot],
                                  out_hbm.at[pl.ds((b0 + bi) * OW, XW)],
                                  xosems[slot]).wait()

        def issue_u(bi, slot):
            pltpu.async_copy(us[slot],
                             out_hbm.at[pl.ds((b0 + bi) * OW + XW, XW)],
                             usems[slot])

        def wait_u(bi, slot):
            pltpu.make_async_copy(us[slot],
                                  out_hbm.at[pl.ds((b0 + bi) * OW + XW, XW)],
                                  usems[slot]).wait()

        def upsample(slot):
            ev = embs[slot]
            uv = us[slot]

            def cloop(c, carry):
                cvec = jnp.full((16,), c, jnp.int32)
                crow = c * S
                for k, (s0, _) in enumerate(chunk_plan):
                    vals = plsc.load_gather(ev, [bases[k], cvec])
                    uv[pl.ds(crow + s0, 16)] = vals
                return carry

            lax.fori_loop(0, C, cloop, 0)

        issue_gather(0, 0)
        issue_xin(0, 0)
        issue_gather(1, 1)
        issue_xin(1, 1)

        def half(bp, bi, slot):
            wait_xin(bi, slot)
            issue_xout(bi, slot)
            wait_gather(bi, slot)

            @pl.when(bp >= 1)
            def _():
                wait_u(bi, slot)

            upsample(slot)
            issue_u(bi, slot)

            @pl.when(bi + 2 < PB)
            def _():
                wait_xout(bi, slot)
                issue_gather(bi + 2, slot)
                issue_xin(bi + 2, slot)

        def pair(bp, carry):
            half(bp, 2 * bp, 0)
            half(bp, 2 * bp + 1, 1)
            return carry

        lax.fori_loop(0, PB // 2, pair, 0)

        wait_u(PB - 2, 0)
        wait_u(PB - 1, 1)
        wait_xout(PB - 2, 0)
        wait_xout(PB - 1, 1)

    return body(x_flat, codes_p, table)


def kernel(x, codes, W_emb):
    b, c, S = x.shape
    _, N = codes.shape

    info = plsc.get_sparse_core_info()
    NC, NS = info.num_cores, info.num_subcores
    NW = NC * NS

    NP = 56
    codes_p = jnp.pad(codes.astype(jnp.int32), ((0, 0), (0, NP - N)))

    out_flat = _sc_block(x.reshape(-1), codes_p.reshape(-1), W_emb,
                         b, c, S, NC, NS)
    return out_flat.reshape(b, 2 * c, S)
